# baseline (speedup 1.0000x reference)
import functools

import jax
import jax.numpy as jnp
from jax import lax
from jax.experimental import pallas as pl
from jax.experimental.pallas import tpu as pltpu
from jax.experimental.pallas import tpu_sc as plsc

_EPS = 1e-12


def _sc_gather_stats(word_embeddings, flat_ids, pos_flat, tok_flat, b, s, hid):
    info = plsc.get_sparse_core_info()
    nw = info.num_cores * info.num_subcores
    n = b * s
    ppw = s // nw
    half = ppw // 2
    n_chunks = 2 * b
    mesh = plsc.VectorSubcoreMesh(core_axis_name="c", subcore_axis_name="s")
    nvec = hid // 16

    @functools.partial(
        pl.kernel,
        mesh=mesh,
        out_type=(
            jax.ShapeDtypeStruct((n, hid), jnp.float32),
            jax.ShapeDtypeStruct((n, 32), jnp.float32),
        ),
        scratch_types=[
            pltpu.VMEM((b * ppw,), jnp.int32),
            pltpu.VMEM((half, hid), jnp.float32),
            pltpu.VMEM((half, hid), jnp.float32),
            pltpu.VMEM((half, hid), jnp.float32),
            pltpu.VMEM((hid,), jnp.float32),
            pltpu.VMEM((b * ppw, 32), jnp.float32),
            pltpu.SemaphoreType.DMA,
            pltpu.SemaphoreType.DMA,
        ],
    )
    def k(table_hbm, idx_hbm, pos_hbm, tok_hbm, x_hbm, st_hbm,
          idxb, gbuf0, gbuf1, pbuf, tokb, sbuf, gsem, wsem):
        wid = lax.axis_index("s") * info.num_cores + lax.axis_index("c")
        p0 = wid * ppw

        def decode(c):
            return c // b, c % b

        for c in range(n_chunks):
            h, bb = decode(c)
            pltpu.sync_copy(idx_hbm.at[pl.ds(bb * s + p0 + h * half, half)],
                            idxb.at[pl.ds(c * half, half)])
        pltpu.sync_copy(tok_hbm.at[pl.ds(0, hid)], tokb)

        def prefill(h):
            pltpu.sync_copy(pos_hbm.at[pl.ds(p0 + h * half, half)], pbuf)

            def tokadd(t, carry):
                for kk in range(nvec):
                    sl = pl.ds(16 * kk, 16)
                    pbuf[t, sl] = pbuf[t, sl] + tokb[sl]
                return carry
            lax.fori_loop(0, half, tokadd, 0)

        gbufs = (gbuf0, gbuf1)

        def start_gather(c):
            return pltpu.async_copy(
                table_hbm.at[idxb.at[pl.ds(c * half, half)]],
                gbufs[c % 2], gsem)

        def flat0(c):
            h, bb = decode(c)
            return bb * s + p0 + h * half

        def compute(c, gbuf):
            h, bb = decode(c)
            srow0 = bb * ppw + h * half

            def body(t, carry):
                s1 = jnp.zeros((16,), jnp.float32)
                s2 = jnp.zeros((16,), jnp.float32)
                for kk in range(nvec):
                    sl = pl.ds(16 * kk, 16)
                    xv = gbuf[t, sl] + pbuf[t, sl]
                    gbuf[t, sl] = xv
                    s1 = s1 + xv
                    s2 = s2 + xv * xv
                sbuf[srow0 + t, pl.ds(0, 16)] = s1
                sbuf[srow0 + t, pl.ds(16, 16)] = s2
                return carry
            lax.fori_loop(0, half, body, 0)

        g = [start_gather(0), start_gather(1)]
        wb = [None, None]
        prefill(0)
        for c in range(n_chunks):
            p = c % 2
            if c == b:
                prefill(1)
            g[p].wait()
            compute(c, gbufs[p])
            wb[p] = pltpu.async_copy(gbufs[p], x_hbm.at[pl.ds(flat0(c), half)], wsem)
            if c + 2 < n_chunks:
                wb[p].wait()
                g[p] = start_gather(c + 2)
        for c in range(n_chunks - 2, n_chunks):
            wb[c % 2].wait()

        ws = []
        for bb in range(b):
            ws.append(pltpu.async_copy(
                sbuf.at[pl.ds(bb * ppw, ppw)],
                st_hbm.at[pl.ds(bb * s + p0, ppw)], wsem))
        for w_ in ws:
            w_.wait()

    return k(word_embeddings, flat_ids, pos_flat, tok_flat)


def _apply_body(x_ref, st_ref, w_ref, b_ref, o_ref):
    hid = x_ref.shape[-1]
    st = st_ref[0]
    s1 = jnp.sum(st[:, 0:16], axis=-1, keepdims=True)
    s2 = jnp.sum(st[:, 16:32], axis=-1, keepdims=True)
    mean = s1 * (1.0 / hid)
    var = s2 * (1.0 / hid) - mean * mean
    inv = lax.rsqrt(var + _EPS)
    o_ref[0] = (x_ref[0] - mean) * inv * w_ref[...] + b_ref[...]


def kernel(input_ids, word_embeddings, position_embeddings, token_type_embeddings, ln_weight, ln_bias):
    b, s = input_ids.shape
    vocab, hid = word_embeddings.shape
    n = b * s
    flat_ids = input_ids.reshape(n).astype(jnp.int32)

    x, st = _sc_gather_stats(
        word_embeddings, flat_ids, position_embeddings,
        token_type_embeddings.reshape(-1), b, s, hid)
    x = x.reshape(b, s, hid)
    st = st.reshape(b, s, 32)

    bs = 2048
    out = pl.pallas_call(
        _apply_body,
        grid=(b, s // bs),
        in_specs=[
            pl.BlockSpec((1, bs, hid), lambda i, j: (i, j, 0)),
            pl.BlockSpec((1, bs, 32), lambda i, j: (i, j, 0)),
            pl.BlockSpec((1, hid), lambda i, j: (0, 0)),
            pl.BlockSpec((1, hid), lambda i, j: (0, 0)),
        ],
        out_specs=pl.BlockSpec((1, bs, hid), lambda i, j: (i, j, 0)),
        out_shape=jax.ShapeDtypeStruct((b, s, hid), jnp.float32),
    )(x, st, ln_weight.reshape(1, hid), ln_bias.reshape(1, hid))
    return out


# speedup vs baseline: 1.0807x; 1.0807x over previous
import functools

import jax
import jax.numpy as jnp
from jax import lax
from jax.experimental import pallas as pl
from jax.experimental.pallas import tpu as pltpu
from jax.experimental.pallas import tpu_sc as plsc

_EPS = 1e-12
_Q = 4


def _sc_gather_slice(word_embeddings, flat_ids, q, b, s, hid):
    info = plsc.get_sparse_core_info()
    nw = info.num_cores * info.num_subcores
    sl = s // _Q
    n_tok = b * sl
    per_w = n_tok // nw
    chunk = per_w // 2
    wps = sl // per_w
    mesh = plsc.VectorSubcoreMesh(core_axis_name="c", subcore_axis_name="s")

    @functools.partial(
        pl.kernel,
        mesh=mesh,
        out_type=jax.ShapeDtypeStruct((n_tok, hid), jnp.float32),
        scratch_types=[
            pltpu.VMEM((per_w,), jnp.int32),
            pltpu.VMEM((chunk, hid), jnp.float32),
            pltpu.VMEM((chunk, hid), jnp.float32),
            pltpu.SemaphoreType.DMA,
            pltpu.SemaphoreType.DMA,
        ],
    )
    def gather_k(table_hbm, idx_hbm, out_hbm, idx_v, buf0, buf1, gsem, wsem):
        wid = lax.axis_index("s") * info.num_cores + lax.axis_index("c")
        bb = wid // wps
        pl0 = (wid % wps) * per_w
        src0 = bb * s + q * sl + pl0
        dst0 = bb * sl + pl0
        bufs = (buf0, buf1)
        pltpu.sync_copy(idx_hbm.at[pl.ds(src0, per_w)], idx_v)

        def start_gather(i):
            return pltpu.async_copy(
                table_hbm.at[idx_v.at[pl.ds(i * chunk, chunk)]], bufs[i % 2], gsem
            )

        gathers = [start_gather(0), start_gather(1)]
        writes = [None, None]
        n_chunks = per_w // chunk
        for i in range(n_chunks):
            gathers[i % 2].wait()
            writes[i % 2] = pltpu.async_copy(
                bufs[i % 2], out_hbm.at[pl.ds(dst0 + i * chunk, chunk)], wsem
            )
            if i + 2 < n_chunks:
                writes[i % 2].wait()
                gathers[i % 2] = start_gather(i + 2)
        writes[(n_chunks - 2) % 2].wait()
        writes[(n_chunks - 1) % 2].wait()

    return gather_k(word_embeddings, flat_ids)


def _ln_body(g_ref, pos_ref, tok_ref, w_ref, b_ref, obuf_ref, o_ref):
    x = g_ref[0] + pos_ref[...] + tok_ref[...]
    mean = jnp.mean(x, axis=-1, keepdims=True)
    xc = x - mean
    var = jnp.mean(xc * xc, axis=-1, keepdims=True)
    o_ref[0] = (xc * lax.rsqrt(var + _EPS)) * w_ref[...] + b_ref[...]


def kernel(input_ids, word_embeddings, position_embeddings, token_type_embeddings, ln_weight, ln_bias):
    b, s = input_ids.shape
    vocab, hid = word_embeddings.shape
    n = b * s
    sl = s // _Q
    flat_ids = input_ids.reshape(n).astype(jnp.int32)

    gathered = [
        _sc_gather_slice(word_embeddings, flat_ids, q, b, s, hid).reshape(b, sl, hid)
        for q in range(_Q)
    ]

    tok = token_type_embeddings[0:1]
    w2 = ln_weight.reshape(1, hid)
    b2 = ln_bias.reshape(1, hid)

    bs = 256
    out = jnp.zeros((b, s, hid), jnp.float32)
    for q in range(_Q):
        pos_q = lax.slice_in_dim(position_embeddings, q * sl, (q + 1) * sl, axis=0)
        out = pl.pallas_call(
            _ln_body,
            grid=(b, sl // bs),
            in_specs=[
                pl.BlockSpec((1, bs, hid), lambda i, j: (i, j, 0)),
                pl.BlockSpec((bs, hid), lambda i, j: (j, 0)),
                pl.BlockSpec((1, hid), lambda i, j: (0, 0)),
                pl.BlockSpec((1, hid), lambda i, j: (0, 0)),
                pl.BlockSpec((1, hid), lambda i, j: (0, 0)),
                pl.BlockSpec(memory_space=pl.ANY),
            ],
            out_specs=pl.BlockSpec(
                (1, bs, hid), lambda i, j, q=q: (i, q * (sl // bs) + j, 0)
            ),
            out_shape=jax.ShapeDtypeStruct((b, s, hid), jnp.float32),
            input_output_aliases={5: 0},
        )(gathered[q], pos_q, tok, w2, b2, out)
    return out


# speedup vs baseline: 1.6232x; 1.5020x over previous
import functools

import jax
import jax.numpy as jnp
from jax import lax
from jax.experimental import pallas as pl
from jax.experimental.pallas import tpu as pltpu
from jax.experimental.pallas import tpu_sc as plsc

_EPS = 1e-12


def _sc_gather(word_embeddings, flat_ids, n_tokens, hid):
    info = plsc.get_sparse_core_info()
    nw = info.num_cores * info.num_subcores
    per_w = n_tokens // nw
    chunk = 64
    n_chunks = per_w // chunk
    mesh = plsc.VectorSubcoreMesh(core_axis_name="c", subcore_axis_name="s")

    @functools.partial(
        pl.kernel,
        mesh=mesh,
        out_type=jax.ShapeDtypeStruct((n_tokens, hid), jnp.float32),
        scratch_types=[
            pltpu.VMEM((per_w,), jnp.int32),
            pltpu.VMEM((chunk, hid), jnp.float32),
            pltpu.VMEM((chunk, hid), jnp.float32),
            pltpu.SemaphoreType.DMA,
            pltpu.SemaphoreType.DMA,
        ],
    )
    def gather_k(table_hbm, idx_hbm, out_hbm, idx_v, buf0, buf1, gsem, wsem):
        wid = lax.axis_index("s") * info.num_cores + lax.axis_index("c")
        base = wid * per_w
        bufs = (buf0, buf1)
        pltpu.sync_copy(idx_hbm.at[pl.ds(base, per_w)], idx_v)

        def start_gather(i):
            return pltpu.async_copy(
                table_hbm.at[idx_v.at[pl.ds(i * chunk, chunk)]], bufs[i % 2], gsem
            )

        gathers = [start_gather(0), start_gather(1)]
        writes = [None, None]
        for i in range(n_chunks):
            gathers[i % 2].wait()
            writes[i % 2] = pltpu.async_copy(
                bufs[i % 2], out_hbm.at[pl.ds(base + i * chunk, chunk)], wsem
            )
            if i + 2 < n_chunks:
                writes[i % 2].wait()
                gathers[i % 2] = start_gather(i + 2)
        writes[(n_chunks - 2) % 2].wait()
        writes[(n_chunks - 1) % 2].wait()

    return gather_k(word_embeddings, flat_ids)


def _ln_body(g_ref, pos_ref, tok_ref, w_ref, b_ref, o_ref):
    x = g_ref[0] + pos_ref[...] + tok_ref[...]
    mean = jnp.mean(x, axis=-1, keepdims=True)
    xc = x - mean
    var = jnp.mean(xc * xc, axis=-1, keepdims=True)
    o_ref[0] = (xc * lax.rsqrt(var + _EPS)) * w_ref[...] + b_ref[...]


def kernel(input_ids, word_embeddings, position_embeddings, token_type_embeddings, ln_weight, ln_bias):
    b, s = input_ids.shape
    vocab, hid = word_embeddings.shape
    n_tokens = b * s
    flat_ids = input_ids.reshape(n_tokens).astype(jnp.int32)

    gathered = _sc_gather(word_embeddings, flat_ids, n_tokens, hid)
    gathered = gathered.reshape(b, s, hid)

    bs = 2048
    out = pl.pallas_call(
        _ln_body,
        grid=(b, s // bs),
        in_specs=[
            pl.BlockSpec((1, bs, hid), lambda i, j: (i, j, 0)),
            pl.BlockSpec((bs, hid), lambda i, j: (j, 0)),
            pl.BlockSpec((1, hid), lambda i, j: (0, 0)),
            pl.BlockSpec((1, hid), lambda i, j: (0, 0)),
            pl.BlockSpec((1, hid), lambda i, j: (0, 0)),
        ],
        out_specs=pl.BlockSpec((1, bs, hid), lambda i, j: (i, j, 0)),
        out_shape=jax.ShapeDtypeStruct((b, s, hid), jnp.float32),
    )(
        gathered,
        position_embeddings,
        token_type_embeddings[0:1],
        ln_weight.reshape(1, hid),
        ln_bias.reshape(1, hid),
    )
    return out
